# baseline (speedup 1.0000x reference)
import functools

import jax
import jax.numpy as jnp
from jax import lax
from jax.experimental import pallas as pl
from jax.experimental.pallas import tpu as pltpu
from jax.experimental.pallas import tpu_sc as plsc

NC = 2
NS = 16
NW = NC * NS
BPC = 2
NBUF = 4
AHEAD = 3
SB = 2
LANES = 16
VB = 1024


def _prep_kernel(xt_ref, out_ref):
    dim = xt_ref.shape[0]
    eye = (lax.broadcasted_iota(jnp.int32, (dim, dim), 0) ==
           lax.broadcasted_iota(jnp.int32, (dim, dim), 1)).astype(jnp.float32)
    out_ref[:, :dim] = lax.dot_general(
        xt_ref[...], eye, (((0,), (0,)), ((), ())),
        precision=lax.Precision.HIGHEST)


@jax.jit
def _prep(embt):
    dim, vocab = embt.shape
    grid = pl.cdiv(vocab, VB)
    return pl.pallas_call(
        _prep_kernel,
        grid=(grid,),
        in_specs=[pl.BlockSpec((dim, VB), lambda i: (0, i))],
        out_specs=pl.BlockSpec((VB, 2 * dim), lambda i: (i, 0)),
        out_shape=jax.ShapeDtypeStruct((vocab, 2 * dim), jnp.float32),
    )(embt)


@functools.partial(jax.jit, static_argnums=(2, 3, 4))
def _sc_embed(embp, idx3, batch, hist, dim):
    n_chunks = idx3.shape[1]
    bpw = n_chunks * BPC
    mesh = plsc.VectorSubcoreMesh(core_axis_name="c", subcore_axis_name="s")

    @functools.partial(
        pl.kernel,
        mesh=mesh,
        out_type=jax.ShapeDtypeStruct((batch, hist, dim), jnp.float32),
        scratch_types=[
            pltpu.VMEM((n_chunks, BPC * hist), jnp.int32),
            pltpu.VMEM((NBUF, BPC * hist, 2 * dim), jnp.float32),
            pltpu.VMEM((SB, BPC, hist, dim), jnp.float32),
            pltpu.SemaphoreType.DMA((NBUF,)),
            pltpu.SemaphoreType.DMA((SB, BPC)),
        ],
        compiler_params=pltpu.CompilerParams(use_tc_tiling_on_sc=True),
    )
    def k(table_hbm, idx_hbm, out_hbm, idx_v, rows_v, sbuf, gsem, ssem):
        wid = lax.axis_index("s") * NC + lax.axis_index("c")
        pltpu.sync_copy(idx_hbm.at[wid], idx_v)

        def gather_desc(g, b):
            return pltpu.make_async_copy(
                table_hbm.at[idx_v.at[g]], rows_v.at[b], gsem.at[b])

        def store_desc(g, sb, j):
            return pltpu.make_async_copy(
                sbuf.at[sb, j], out_hbm.at[wid * bpw + g * BPC + j],
                ssem.at[sb, j])

        for g0 in range(AHEAD):
            gather_desc(g0, g0).start()

        def compact(b, sb):
            for j in range(BPC):
                for h in range(hist):
                    for l in range(dim // LANES):
                        sbuf[sb, j, h, pl.ds(l * LANES, LANES)] = (
                            rows_v[b, j * hist + h, pl.ds(l * LANES, LANES)])

        def body(g, _):
            b = lax.rem(g, NBUF)
            gn = g + AHEAD
            bn = lax.rem(gn, NBUF)

            @pl.when(gn < n_chunks)
            def _():
                gather_desc(gn, bn).start()

            gather_desc(g, b).wait()

            sb = lax.rem(g, SB)

            @pl.when(g >= SB)
            def _():
                for j in range(BPC):
                    store_desc(g - SB, sb, j).wait()

            compact(b, sb)
            for j in range(BPC):
                store_desc(g, sb, j).start()
            return 0

        lax.fori_loop(0, n_chunks, body, 0, unroll=False)

        for c in range(n_chunks - SB, n_chunks):
            for j in range(BPC):
                store_desc(c, c % SB, j).wait()

    return k(embp, idx3)


def kernel(emb, idxs):
    batch, hist = idxs.shape
    vocab, dim = emb.shape
    bpw = batch // NW
    n_chunks = bpw // BPC
    embp = _prep(emb.T)
    idx3 = idxs.astype(jnp.int32).reshape(NW, n_chunks, BPC * hist)
    return _sc_embed(embp, idx3, batch, hist, dim)


# speedup vs baseline: 1.2906x; 1.2906x over previous
import functools

import jax
import jax.numpy as jnp
from jax import lax
from jax.experimental import pallas as pl
from jax.experimental.pallas import tpu as pltpu
from jax.experimental.pallas import tpu_sc as plsc

NC = 2
NS = 16
NW = NC * NS
BPC = 2
NBUF = 4
AHEAD = 3
SB = 2
LANES = 16


@functools.partial(jax.jit, static_argnums=(2, 3, 4))
def _sc_embed(embp, idx3, batch, hist, dim):
    n_chunks = idx3.shape[1]
    bpw = n_chunks * BPC
    mesh = plsc.VectorSubcoreMesh(core_axis_name="c", subcore_axis_name="s")

    @functools.partial(
        pl.kernel,
        mesh=mesh,
        out_type=jax.ShapeDtypeStruct((batch, hist, dim), jnp.float32),
        scratch_types=[
            pltpu.VMEM((n_chunks, BPC * hist), jnp.int32),
            pltpu.VMEM((NBUF, BPC * hist, 2 * dim), jnp.float32),
            pltpu.VMEM((SB, BPC, hist, dim), jnp.float32),
            pltpu.SemaphoreType.DMA((NBUF,)),
            pltpu.SemaphoreType.DMA((SB, BPC)),
        ],
        compiler_params=pltpu.CompilerParams(use_tc_tiling_on_sc=True),
    )
    def k(table_hbm, idx_hbm, out_hbm, idx_v, rows_v, sbuf, gsem, ssem):
        wid = lax.axis_index("s") * NC + lax.axis_index("c")
        pltpu.sync_copy(idx_hbm.at[wid], idx_v)

        def gather_desc(g, b):
            return pltpu.make_async_copy(
                table_hbm.at[idx_v.at[g]], rows_v.at[b], gsem.at[b])

        def store_desc(g, sb, j):
            return pltpu.make_async_copy(
                sbuf.at[sb, j], out_hbm.at[wid * bpw + g * BPC + j],
                ssem.at[sb, j])

        for g0 in range(AHEAD):
            gather_desc(g0, g0).start()

        def compact(b, sb):
            for j in range(BPC):
                for h in range(hist):
                    for l in range(dim // LANES):
                        sbuf[sb, j, h, pl.ds(l * LANES, LANES)] = (
                            rows_v[b, j * hist + h, pl.ds(l * LANES, LANES)])

        def body(g, _):
            b = lax.rem(g, NBUF)
            gn = g + AHEAD
            bn = lax.rem(gn, NBUF)

            @pl.when(gn < n_chunks)
            def _():
                gather_desc(gn, bn).start()

            gather_desc(g, b).wait()

            sb = lax.rem(g, SB)

            @pl.when(g >= SB)
            def _():
                for j in range(BPC):
                    store_desc(g - SB, sb, j).wait()

            compact(b, sb)
            for j in range(BPC):
                store_desc(g, sb, j).start()
            return 0

        lax.fori_loop(0, n_chunks, body, 0, unroll=False)

        for c in range(n_chunks - SB, n_chunks):
            for j in range(BPC):
                store_desc(c, c % SB, j).wait()

    return k(embp, idx3)


def kernel(emb, idxs):
    batch, hist = idxs.shape
    vocab, dim = emb.shape
    bpw = batch // NW
    n_chunks = bpw // BPC
    embp = jnp.pad(emb, ((0, 0), (0, dim)))
    idx3 = idxs.astype(jnp.int32).reshape(NW, n_chunks, BPC * hist)
    return _sc_embed(embp, idx3, batch, hist, dim)


# speedup vs baseline: 1.2912x; 1.0004x over previous
import functools

import jax
import jax.numpy as jnp
from jax import lax
from jax.experimental import pallas as pl
from jax.experimental.pallas import tpu as pltpu
from jax.experimental.pallas import tpu_sc as plsc

NC = 2
NS = 16
NW = NC * NS
BPC = 2
NBUF = 3
AHEAD = 2
SB = 3
LANES = 16


@functools.partial(jax.jit, static_argnums=(2, 3, 4))
def _sc_embed(embp, idx3, batch, hist, dim):
    n_chunks = idx3.shape[1]
    bpw = n_chunks * BPC
    mesh = plsc.VectorSubcoreMesh(core_axis_name="c", subcore_axis_name="s")

    @functools.partial(
        pl.kernel,
        mesh=mesh,
        out_type=jax.ShapeDtypeStruct((batch, hist, dim), jnp.float32),
        scratch_types=[
            pltpu.VMEM((n_chunks, BPC * hist), jnp.int32),
            pltpu.VMEM((NBUF, BPC * hist, 2 * dim), jnp.float32),
            pltpu.VMEM((SB, BPC, hist, dim), jnp.float32),
            pltpu.SemaphoreType.DMA((NBUF,)),
            pltpu.SemaphoreType.DMA((SB, BPC)),
        ],
        compiler_params=pltpu.CompilerParams(use_tc_tiling_on_sc=True),
    )
    def k(table_hbm, idx_hbm, out_hbm, idx_v, rows_v, sbuf, gsem, ssem):
        wid = lax.axis_index("s") * NC + lax.axis_index("c")
        pltpu.sync_copy(idx_hbm.at[wid], idx_v)

        def gather_desc(g, b):
            return pltpu.make_async_copy(
                table_hbm.at[idx_v.at[g]], rows_v.at[b], gsem.at[b])

        def store_desc(g, sb, j):
            return pltpu.make_async_copy(
                sbuf.at[sb, j], out_hbm.at[wid * bpw + g * BPC + j],
                ssem.at[sb, j])

        for g0 in range(AHEAD):
            gather_desc(g0, g0).start()

        def compact(b, sb):
            for j in range(BPC):
                for h in range(hist):
                    for l in range(dim // LANES):
                        sbuf[sb, j, h, pl.ds(l * LANES, LANES)] = (
                            rows_v[b, j * hist + h, pl.ds(l * LANES, LANES)])

        def body(g, _):
            b = lax.rem(g, NBUF)
            gn = g + AHEAD
            bn = lax.rem(gn, NBUF)

            @pl.when(gn < n_chunks)
            def _():
                gather_desc(gn, bn).start()

            gather_desc(g, b).wait()

            sb = lax.rem(g, SB)

            @pl.when(g >= SB)
            def _():
                for j in range(BPC):
                    store_desc(g - SB, sb, j).wait()

            compact(b, sb)
            for j in range(BPC):
                store_desc(g, sb, j).start()
            return 0

        lax.fori_loop(0, n_chunks, body, 0, unroll=False)

        for c in range(n_chunks - SB, n_chunks):
            for j in range(BPC):
                store_desc(c, c % SB, j).wait()

    return k(embp, idx3)


def kernel(emb, idxs):
    batch, hist = idxs.shape
    vocab, dim = emb.shape
    bpw = batch // NW
    n_chunks = bpw // BPC
    embp = jnp.pad(emb, ((0, 0), (0, dim)))
    idx3 = idxs.astype(jnp.int32).reshape(NW, n_chunks, BPC * hist)
    return _sc_embed(embp, idx3, batch, hist, dim)
